# baseline (device time: 394723 ns/iter reference)
import jax
import jax.numpy as jnp
from jax import lax
from jax.experimental import pallas as pl
from jax.experimental.pallas import tpu as pltpu

N_DEV = 4
BLK = 2048
CH = 512
N_CH = BLK // CH
K = 2048
N = 2048

_HBM = pltpu.HBM


def kernel(t, W):
    def body(t_ref, w_ref, out_ref, rs_hbm, ag_hbm, t16_hbm,
             cast_in, cast_out, own16, w16, sum_st, y16, ag_in, ag_out,
             rs_recv, ag_recv, rs_send, ag_send, loc):
        me = lax.axis_index("i")
        up = (me + 1) % N_DEV
        dn = (me + 3) % N_DEV
        dg = (me + 2) % N_DEV


        barrier = pltpu.get_barrier_semaphore()
        for off in (1, 2, 3):
            peer = (me + off) % N_DEV
            pl.semaphore_signal(
                barrier, inc=1,
                device_id=(peer,), device_id_type=pl.DeviceIdType.MESH,
            )
        pl.semaphore_wait(barrier, N_DEV - 1)

        rdmas = []

        items = []
        for c in range(N_CH):
            items.append(("diag", c))
        items += [("pure", 0), ("pure", 1), ("pure", 2), ("pure", 3)]
        for c in range(N_CH):
            items.append(("own", c))
        for c in range(K // CH):
            items.append(("w", c))

        def item_info(it):
            kind, c = it
            if kind == "diag":
                row0 = dg * BLK + c * CH
                if c < 2:
                    return row0, row0, up, 2, 2, 3, c
                return row0, row0, dn, 3, 3, 3, c
            if c < 2:
                row0 = dn * BLK + c * CH
                return row0, row0, dn, 1, 1, 2, c
            row0 = up * BLK + c * CH
            return row0, row0, up, 0, 0, 1, c

        def src_slice(it):
            kind, c = it
            if kind == "w":
                return w_ref.at[pl.ds(c * CH, CH), :]
            if kind == "own":
                return t_ref.at[pl.ds(me * BLK + c * CH, CH), :]
            return t_ref.at[pl.ds(item_info(it)[0], CH), :]

        loads = [None] * len(items)

        def start_load(k):
            cp = pltpu.make_async_copy(
                src_slice(items[k]), cast_in.at[k % 2], loc.at[k % 2])
            cp.start()
            loads[k] = cp

        def send_chunk(hbm_row0, dev, dst_slot, recv_row, send_row, c):
            rdma = pltpu.make_async_remote_copy(
                src_ref=t16_hbm.at[pl.ds(hbm_row0, CH), :],
                dst_ref=rs_hbm.at[dst_slot, pl.ds(c * CH, CH), :],
                send_sem=rs_send.at[send_row, c],
                recv_sem=rs_recv.at[recv_row, c],
                device_id=(dev,),
                device_id_type=pl.DeviceIdType.MESH,
            )
            rdma.start()
            rdmas.append(rdma)

        pend = [None, None]

        def flush_slot(s2):
            if pend[s2] is not None:
                st, info = pend[s2]
                st.wait()
                send_chunk(*info)
                pend[s2] = None

        n_st = 0
        start_load(0)
        for k, it in enumerate(items):
            if k + 1 < len(items):
                start_load(k + 1)
            loads[k].wait()
            kind, c = it
            s = k % 2
            if kind == "w":
                w16[pl.ds(c * CH, CH), :] = cast_in[s].astype(jnp.bfloat16)
            elif kind == "own":
                own16[pl.ds(c * CH, CH), :] = cast_in[s].astype(jnp.bfloat16)
            else:
                _, hrow, dev, slot, rrow, srow, cc = item_info(it)
                s2 = n_st % 2
                n_st += 1
                flush_slot(s2)
                cast_out[s2, :, :] = cast_in[s].astype(jnp.bfloat16)
                st = pltpu.make_async_copy(
                    cast_out.at[s2], t16_hbm.at[pl.ds(hrow, CH), :],
                    loc.at[2 + s2])
                st.start()
                pend[s2] = (st, (hrow, dev, slot, rrow, srow, cc))
        flush_slot(0)
        flush_slot(1)

        for c in range(N_CH):
            stage_slot = 2 if c < 2 else 3
            blk = up if c < 2 else dn
            dev = up if c < 2 else dn
            dst_slot = 0 if c < 2 else 1
            send_row = 1 if c < 2 else 2
            pltpu.make_async_remote_copy(
                src_ref=rs_hbm.at[stage_slot, pl.ds(c * CH, CH), :],
                dst_ref=rs_hbm.at[stage_slot, pl.ds(c * CH, CH), :],
                send_sem=rs_send.at[0, 0],
                recv_sem=rs_recv.at[stage_slot, c],
                device_id=(me,),
                device_id_type=pl.DeviceIdType.MESH,
            ).wait_recv()
            ld0 = pltpu.make_async_copy(
                rs_hbm.at[stage_slot, pl.ds(c * CH, CH), :],
                sum_st.at[0], loc.at[0])
            ld0.start()
            row0 = blk * BLK + c * CH
            ld1 = pltpu.make_async_copy(
                t_ref.at[pl.ds(row0, CH), :], cast_in.at[0], loc.at[1])
            ld1.start()
            ld0.wait()
            ld1.wait()
            s2 = n_st % 2
            n_st += 1
            flush_slot(s2)
            cast_out[s2, :, :] = (
                sum_st[0].astype(jnp.float32) + cast_in[0]
            ).astype(jnp.bfloat16)
            st = pltpu.make_async_copy(
                cast_out.at[s2], t16_hbm.at[pl.ds(row0, CH), :],
                loc.at[2 + s2])
            st.start()
            pend[s2] = (st, (row0, dev, dst_slot, dst_slot, send_row, c))
        flush_slot(0)
        flush_slot(1)

        ag_out_cp = [None, None]
        ag_slot = 0
        for c in range(N_CH):
            cps = []
            for slot in (0, 1):
                pltpu.make_async_remote_copy(
                    src_ref=rs_hbm.at[slot, pl.ds(c * CH, CH), :],
                    dst_ref=rs_hbm.at[slot, pl.ds(c * CH, CH), :],
                    send_sem=rs_send.at[0, 0],
                    recv_sem=rs_recv.at[slot, c],
                    device_id=(me,),
                    device_id_type=pl.DeviceIdType.MESH,
                ).wait_recv()
                cp = pltpu.make_async_copy(
                    rs_hbm.at[slot, pl.ds(c * CH, CH), :],
                    sum_st.at[slot], loc.at[slot])
                cp.start()
                cps.append(cp)
            for cp in cps:
                cp.wait()
            s_f32 = (
                sum_st[0].astype(jnp.float32)
                + sum_st[1].astype(jnp.float32)
                + own16[pl.ds(c * CH, CH), :].astype(jnp.float32)
            )
            val = jnp.dot(
                s_f32.astype(jnp.bfloat16), w16[...],
                preferred_element_type=jnp.float32)
            sl = ag_slot % 2
            ag_slot += 1
            if ag_out_cp[sl] is not None:
                ag_out_cp[sl].wait()
            ag_out[sl, :, :] = val
            stc = pltpu.make_async_copy(
                ag_out.at[sl],
                out_ref.at[pl.ds(me * BLK + c * CH, CH), :],
                loc.at[4 + sl])
            stc.start()
            ag_out_cp[sl] = stc
            y16[pl.ds(c * CH, CH), :] = val.astype(jnp.bfloat16)
            for dev, srow in ((up, 1), (dn, 2)):
                rdma = pltpu.make_async_remote_copy(
                    src_ref=y16.at[pl.ds(c * CH, CH), :],
                    dst_ref=ag_hbm.at[me, pl.ds(c * CH, CH), :],
                    send_sem=ag_send.at[srow, c],
                    recv_sem=ag_recv.at[me, c],
                    device_id=(dev,),
                    device_id_type=pl.DeviceIdType.MESH,
                )
                rdma.start()
                rdmas.append(rdma)

        seq = []
        for c in range(N_CH):
            seq.append((up, c, dn if c < 2 else None))
            seq.append((dn, c, up if c >= 2 else None))
            seq.append((dg, c, None))
        prev = None
        kk = 0
        for origin, c, relay_to in seq:
            pltpu.make_async_remote_copy(
                src_ref=ag_hbm.at[origin, pl.ds(c * CH, CH), :],
                dst_ref=ag_hbm.at[origin, pl.ds(c * CH, CH), :],
                send_sem=ag_send.at[0, 0],
                recv_sem=ag_recv.at[origin, c],
                device_id=(me,),
                device_id_type=pl.DeviceIdType.MESH,
            ).wait_recv()
            if relay_to is not None:
                rdma = pltpu.make_async_remote_copy(
                    src_ref=ag_hbm.at[origin, pl.ds(c * CH, CH), :],
                    dst_ref=ag_hbm.at[origin, pl.ds(c * CH, CH), :],
                    send_sem=ag_send.at[3, c],
                    recv_sem=ag_recv.at[origin, c],
                    device_id=(relay_to,),
                    device_id_type=pl.DeviceIdType.MESH,
                )
                rdma.start()
                rdmas.append(rdma)
            cp = pltpu.make_async_copy(
                ag_hbm.at[origin, pl.ds(c * CH, CH), :],
                ag_in.at[kk % 2], loc.at[6 + kk % 2])
            cp.start()
            if prev is not None:
                pcp, psrc, pc, ps = prev
                pcp.wait()
                sl = ag_slot % 2
                ag_slot += 1
                if ag_out_cp[sl] is not None:
                    ag_out_cp[sl].wait()
                ag_out[sl, :, :] = ag_in[ps].astype(jnp.float32)
                stc = pltpu.make_async_copy(
                    ag_out.at[sl],
                    out_ref.at[pl.ds(psrc * BLK + pc * CH, CH), :],
                    loc.at[4 + sl])
                stc.start()
                ag_out_cp[sl] = stc
            prev = (cp, origin, c, kk % 2)
            kk += 1
        if prev is not None:
            pcp, psrc, pc, ps = prev
            pcp.wait()
            sl = ag_slot % 2
            if ag_out_cp[sl] is not None:
                ag_out_cp[sl].wait()
            ag_out[sl, :, :] = ag_in[ps].astype(jnp.float32)
            stc = pltpu.make_async_copy(
                ag_out.at[sl],
                out_ref.at[pl.ds(psrc * BLK + pc * CH, CH), :],
                loc.at[4 + sl])
            stc.start()
            ag_out_cp[sl] = stc

        for cp in ag_out_cp:
            if cp is not None:
                cp.wait()
        for r in rdmas:
            r.wait_send()

    out, _, _, _ = pl.pallas_call(
        body,
        out_shape=[
            jax.ShapeDtypeStruct((N_DEV * BLK, N), jnp.float32),
            jax.ShapeDtypeStruct((N_DEV, BLK, K), jnp.bfloat16),
            jax.ShapeDtypeStruct((N_DEV, BLK, N), jnp.bfloat16),
            jax.ShapeDtypeStruct((N_DEV * BLK, K), jnp.bfloat16),
        ],
        in_specs=[
            pl.BlockSpec(memory_space=_HBM),
            pl.BlockSpec(memory_space=_HBM),
        ],
        out_specs=[pl.BlockSpec(memory_space=_HBM)] * 4,
        scratch_shapes=[
            pltpu.VMEM((2, CH, K), jnp.float32),
            pltpu.VMEM((2, CH, K), jnp.bfloat16),
            pltpu.VMEM((BLK, K), jnp.bfloat16),
            pltpu.VMEM((K, N), jnp.bfloat16),
            pltpu.VMEM((2, CH, K), jnp.bfloat16),
            pltpu.VMEM((BLK, N), jnp.bfloat16),
            pltpu.VMEM((2, CH, N), jnp.bfloat16),
            pltpu.VMEM((2, CH, N), jnp.float32),
            pltpu.SemaphoreType.DMA((N_DEV, N_CH)),
            pltpu.SemaphoreType.DMA((N_DEV, N_CH)),
            pltpu.SemaphoreType.DMA((N_DEV, N_CH)),
            pltpu.SemaphoreType.DMA((N_DEV, N_CH)),
            pltpu.SemaphoreType.DMA((8,)),
        ],
        compiler_params=pltpu.CompilerParams(
            collective_id=0, vmem_limit_bytes=62 * 1024 * 1024),
    )(t, W)
    return out


# device time: 363907 ns/iter; 1.0847x vs baseline; 1.0847x over previous
import jax
import jax.numpy as jnp
from jax import lax
from jax.experimental import pallas as pl
from jax.experimental.pallas import tpu as pltpu

N_DEV = 4
BLK = 2048
CH = 512
N_CH = BLK // CH
K = 2048
N = 2048

_HBM = pltpu.HBM


def kernel(t, W):
    def body(t_ref, w_ref, out_ref, rs_hbm, ag_hbm, t16_hbm,
             cast_in, cast_out, own16, w16, sum_st, y16, ag_in, ag_out,
             rs_recv, ag_recv, rs_send, ag_send, loc):
        me = lax.axis_index("i")
        up = (me + 1) % N_DEV
        dn = (me + 3) % N_DEV
        dg = (me + 2) % N_DEV


        barrier = pltpu.get_barrier_semaphore()
        for off in (1, 2, 3):
            peer = (me + off) % N_DEV
            pl.semaphore_signal(
                barrier, inc=1,
                device_id=(peer,), device_id_type=pl.DeviceIdType.MESH,
            )
        pl.semaphore_wait(barrier, N_DEV - 1)

        rdmas = []

        items = []
        for c in range(N_CH):
            items.append(("diag", c))
        items += [("pure", 0), ("pure", 1), ("pure", 2), ("pure", 3)]
        for c in range(N_CH):
            items.append(("own", c))
        for c in range(K // CH):
            items.append(("w", c))

        def item_info(it):
            kind, c = it
            if kind == "diag":
                row0 = dg * BLK + c * CH
                if c < 2:
                    return row0, row0, up, 2, 2, 3, c
                return row0, row0, dn, 3, 3, 3, c
            if c < 2:
                row0 = dn * BLK + c * CH
                return row0, row0, dn, 1, 1, 2, c
            row0 = up * BLK + c * CH
            return row0, row0, up, 0, 0, 1, c

        def src_slice(it):
            kind, c = it
            if kind == "w":
                return w_ref.at[pl.ds(c * CH, CH), :]
            if kind == "own":
                return t_ref.at[pl.ds(me * BLK + c * CH, CH), :]
            return t_ref.at[pl.ds(item_info(it)[0], CH), :]

        loads = [None] * len(items)

        def start_load(k):
            cp = pltpu.make_async_copy(
                src_slice(items[k]), cast_in.at[k % 2], loc.at[k % 2])
            cp.start()
            loads[k] = cp

        def send_chunk(hbm_row0, dev, dst_slot, recv_row, send_row, c):
            rdma = pltpu.make_async_remote_copy(
                src_ref=t16_hbm.at[pl.ds(hbm_row0, CH), :],
                dst_ref=rs_hbm.at[dst_slot, pl.ds(c * CH, CH), :],
                send_sem=rs_send.at[send_row, c],
                recv_sem=rs_recv.at[recv_row, c],
                device_id=(dev,),
                device_id_type=pl.DeviceIdType.MESH,
            )
            rdma.start()
            rdmas.append(rdma)

        pend = [None, None]

        def flush_slot(s2):
            if pend[s2] is not None:
                st, info = pend[s2]
                st.wait()
                send_chunk(*info)
                pend[s2] = None

        n_st = 0
        start_load(0)
        for k, it in enumerate(items):
            if k + 1 < len(items):
                start_load(k + 1)
            loads[k].wait()
            kind, c = it
            s = k % 2
            if kind == "w":
                w16[pl.ds(c * CH, CH), :] = cast_in[s].astype(jnp.bfloat16)
            elif kind == "own":
                own16[pl.ds(c * CH, CH), :] = cast_in[s].astype(jnp.bfloat16)
            else:
                _, hrow, dev, slot, rrow, srow, cc = item_info(it)
                s2 = n_st % 2
                n_st += 1
                flush_slot(s2)
                cast_out[s2, :, :] = cast_in[s].astype(jnp.bfloat16)
                st = pltpu.make_async_copy(
                    cast_out.at[s2], t16_hbm.at[pl.ds(hrow, CH), :],
                    loc.at[2 + s2])
                st.start()
                pend[s2] = (st, (hrow, dev, slot, rrow, srow, cc))
        flush_slot(0)
        flush_slot(1)

        for c in range(N_CH):
            stage_slot = 2 if c < 2 else 3
            blk = up if c < 2 else dn
            dev = up if c < 2 else dn
            dst_slot = 0 if c < 2 else 1
            send_row = 1 if c < 2 else 2
            pltpu.make_async_remote_copy(
                src_ref=rs_hbm.at[stage_slot, pl.ds(c * CH, CH), :],
                dst_ref=rs_hbm.at[stage_slot, pl.ds(c * CH, CH), :],
                send_sem=rs_send.at[0, 0],
                recv_sem=rs_recv.at[stage_slot, c],
                device_id=(me,),
                device_id_type=pl.DeviceIdType.MESH,
            ).wait_recv()
            ld0 = pltpu.make_async_copy(
                rs_hbm.at[stage_slot, pl.ds(c * CH, CH), :],
                sum_st.at[0], loc.at[0])
            ld0.start()
            row0 = blk * BLK + c * CH
            ld1 = pltpu.make_async_copy(
                t_ref.at[pl.ds(row0, CH), :], cast_in.at[0], loc.at[1])
            ld1.start()
            ld0.wait()
            ld1.wait()
            s2 = n_st % 2
            n_st += 1
            flush_slot(s2)
            cast_out[s2, :, :] = (
                sum_st[0].astype(jnp.float32) + cast_in[0]
            ).astype(jnp.bfloat16)
            st = pltpu.make_async_copy(
                cast_out.at[s2], t16_hbm.at[pl.ds(row0, CH), :],
                loc.at[2 + s2])
            st.start()
            pend[s2] = (st, (row0, dev, dst_slot, dst_slot, send_row, c))
        flush_slot(0)
        flush_slot(1)

        ag_out_cp = [None, None]
        ag_slot = 0
        for c in range(N_CH):
            cps = []
            for slot in (0, 1):
                pltpu.make_async_remote_copy(
                    src_ref=rs_hbm.at[slot, pl.ds(c * CH, CH), :],
                    dst_ref=rs_hbm.at[slot, pl.ds(c * CH, CH), :],
                    send_sem=rs_send.at[0, 0],
                    recv_sem=rs_recv.at[slot, c],
                    device_id=(me,),
                    device_id_type=pl.DeviceIdType.MESH,
                ).wait_recv()
                cp = pltpu.make_async_copy(
                    rs_hbm.at[slot, pl.ds(c * CH, CH), :],
                    sum_st.at[slot], loc.at[slot])
                cp.start()
                cps.append(cp)
            for cp in cps:
                cp.wait()
            s_f32 = (
                sum_st[0].astype(jnp.float32)
                + sum_st[1].astype(jnp.float32)
                + own16[pl.ds(c * CH, CH), :].astype(jnp.float32)
            )
            val = jnp.dot(
                s_f32.astype(jnp.bfloat16), w16[...],
                preferred_element_type=jnp.float32)
            y16[pl.ds(c * CH, CH), :] = val.astype(jnp.bfloat16)
            for dev, srow in ((up, 1), (dn, 2)):
                rdma = pltpu.make_async_remote_copy(
                    src_ref=y16.at[pl.ds(c * CH, CH), :],
                    dst_ref=ag_hbm.at[me, pl.ds(c * CH, CH), :],
                    send_sem=ag_send.at[srow, c],
                    recv_sem=ag_recv.at[me, c],
                    device_id=(dev,),
                    device_id_type=pl.DeviceIdType.MESH,
                )
                rdma.start()
                rdmas.append(rdma)
            sl = ag_slot % 2
            ag_slot += 1
            if ag_out_cp[sl] is not None:
                ag_out_cp[sl].wait()
            ag_out[sl, :, :] = val
            stc = pltpu.make_async_copy(
                ag_out.at[sl],
                out_ref.at[pl.ds(me * BLK + c * CH, CH), :],
                loc.at[4 + sl])
            stc.start()
            ag_out_cp[sl] = stc

        seq = []
        for c in range(N_CH):
            seq.append((up, c, dn if c < 2 else None))
            seq.append((dn, c, up if c >= 2 else None))
            if c >= 1:
                seq.append((dg, c - 1, None))
        seq.append((dg, N_CH - 1, None))
        prev = None
        kk = 0
        for origin, c, relay_to in seq:
            pltpu.make_async_remote_copy(
                src_ref=ag_hbm.at[origin, pl.ds(c * CH, CH), :],
                dst_ref=ag_hbm.at[origin, pl.ds(c * CH, CH), :],
                send_sem=ag_send.at[0, 0],
                recv_sem=ag_recv.at[origin, c],
                device_id=(me,),
                device_id_type=pl.DeviceIdType.MESH,
            ).wait_recv()
            if relay_to is not None:
                rdma = pltpu.make_async_remote_copy(
                    src_ref=ag_hbm.at[origin, pl.ds(c * CH, CH), :],
                    dst_ref=ag_hbm.at[origin, pl.ds(c * CH, CH), :],
                    send_sem=ag_send.at[3, c],
                    recv_sem=ag_recv.at[origin, c],
                    device_id=(relay_to,),
                    device_id_type=pl.DeviceIdType.MESH,
                )
                rdma.start()
                rdmas.append(rdma)
            cp = pltpu.make_async_copy(
                ag_hbm.at[origin, pl.ds(c * CH, CH), :],
                ag_in.at[kk % 2], loc.at[6 + kk % 2])
            cp.start()
            if prev is not None:
                pcp, psrc, pc, ps = prev
                pcp.wait()
                sl = ag_slot % 2
                ag_slot += 1
                if ag_out_cp[sl] is not None:
                    ag_out_cp[sl].wait()
                ag_out[sl, :, :] = ag_in[ps].astype(jnp.float32)
                stc = pltpu.make_async_copy(
                    ag_out.at[sl],
                    out_ref.at[pl.ds(psrc * BLK + pc * CH, CH), :],
                    loc.at[4 + sl])
                stc.start()
                ag_out_cp[sl] = stc
            prev = (cp, origin, c, kk % 2)
            kk += 1
        if prev is not None:
            pcp, psrc, pc, ps = prev
            pcp.wait()
            sl = ag_slot % 2
            if ag_out_cp[sl] is not None:
                ag_out_cp[sl].wait()
            ag_out[sl, :, :] = ag_in[ps].astype(jnp.float32)
            stc = pltpu.make_async_copy(
                ag_out.at[sl],
                out_ref.at[pl.ds(psrc * BLK + pc * CH, CH), :],
                loc.at[4 + sl])
            stc.start()
            ag_out_cp[sl] = stc

        for cp in ag_out_cp:
            if cp is not None:
                cp.wait()
        for r in rdmas:
            r.wait_send()

    out, _, _, _ = pl.pallas_call(
        body,
        out_shape=[
            jax.ShapeDtypeStruct((N_DEV * BLK, N), jnp.float32),
            jax.ShapeDtypeStruct((N_DEV, BLK, K), jnp.bfloat16),
            jax.ShapeDtypeStruct((N_DEV, BLK, N), jnp.bfloat16),
            jax.ShapeDtypeStruct((N_DEV * BLK, K), jnp.bfloat16),
        ],
        in_specs=[
            pl.BlockSpec(memory_space=_HBM),
            pl.BlockSpec(memory_space=_HBM),
        ],
        out_specs=[pl.BlockSpec(memory_space=_HBM)] * 4,
        scratch_shapes=[
            pltpu.VMEM((2, CH, K), jnp.float32),
            pltpu.VMEM((2, CH, K), jnp.bfloat16),
            pltpu.VMEM((BLK, K), jnp.bfloat16),
            pltpu.VMEM((K, N), jnp.bfloat16),
            pltpu.VMEM((2, CH, K), jnp.bfloat16),
            pltpu.VMEM((BLK, N), jnp.bfloat16),
            pltpu.VMEM((2, CH, N), jnp.bfloat16),
            pltpu.VMEM((2, CH, N), jnp.float32),
            pltpu.SemaphoreType.DMA((N_DEV, N_CH)),
            pltpu.SemaphoreType.DMA((N_DEV, N_CH)),
            pltpu.SemaphoreType.DMA((N_DEV, N_CH)),
            pltpu.SemaphoreType.DMA((N_DEV, N_CH)),
            pltpu.SemaphoreType.DMA((8,)),
        ],
        compiler_params=pltpu.CompilerParams(
            collective_id=0, vmem_limit_bytes=62 * 1024 * 1024),
    )(t, W)
    return out
